# Initial kernel scaffold; baseline (speedup 1.0000x reference)
import jax
import jax.numpy as jnp
from jax.experimental import pallas as pl


def kernel(assign_logits, codebooks):
    raise NotImplementedError("write your pallas kernel here")


# speedup vs baseline: 1.8866x; 1.8866x over previous
import functools

import jax
import jax.numpy as jnp
from jax.experimental import pallas as pl
from jax.experimental.pallas import tpu as pltpu

V, D, M, K = 50000, 512, 4, 512
CHUNK = D // M


def _dpq_body(x_ref, cb_ref, o_ref):
    x = x_ref[:]
    for m in range(M):
        xm = x[:, m * K:(m + 1) * K]
        mx = jnp.max(xm, axis=-1, keepdims=True)
        e = jnp.exp(xm - mx)
        s = jnp.sum(e, axis=-1, keepdims=True)
        attn = e / s
        cbm = cb_ref[m * K:(m + 1) * K, :]
        o_ref[:, m * CHUNK:(m + 1) * CHUNK] = jax.lax.dot_general(
            attn, cbm, (((1,), (0,)), ((), ())),
            preferred_element_type=jnp.float32,
        )


@functools.partial(jax.jit, static_argnames=("tile_v",))
def _dpq(assign_logits, codebooks, tile_v=1000):
    logits2d = assign_logits.reshape(V, M * K)
    cb2d = codebooks.reshape(M * K, CHUNK)
    grid = (V // tile_v,)
    return pl.pallas_call(
        _dpq_body,
        grid=grid,
        in_specs=[
            pl.BlockSpec((tile_v, M * K), lambda i: (i, 0)),
            pl.BlockSpec((M * K, CHUNK), lambda i: (0, 0)),
        ],
        out_specs=pl.BlockSpec((tile_v, D), lambda i: (i, 0)),
        out_shape=jax.ShapeDtypeStruct((V, D), jnp.float32),
        compiler_params=pltpu.CompilerParams(
            dimension_semantics=("parallel",),
        ),
    )(logits2d, cb2d)


def kernel(assign_logits, codebooks):
    return _dpq(assign_logits, codebooks)


# speedup vs baseline: 1.9056x; 1.0101x over previous
import functools

import jax
import jax.numpy as jnp
from jax.experimental import pallas as pl
from jax.experimental.pallas import tpu as pltpu

V, D, M, K = 50000, 512, 4, 512
CHUNK = D // M


def _dpq_body(x_ref, cb_ref, o_ref):
    x = x_ref[:]
    for m in range(M):
        xm = x[:, m * K:(m + 1) * K]
        mx = jnp.max(xm, axis=-1, keepdims=True)
        e = jnp.exp(xm - mx)
        s = jnp.sum(e, axis=-1, keepdims=True)
        cbm = cb_ref[m * K:(m + 1) * K, :]
        acc = jax.lax.dot_general(
            e.astype(jnp.bfloat16), cbm.astype(jnp.bfloat16),
            (((1,), (0,)), ((), ())),
            preferred_element_type=jnp.float32,
        )
        o_ref[:, m * CHUNK:(m + 1) * CHUNK] = acc * (1.0 / s)


@functools.partial(jax.jit, static_argnames=("tile_v",))
def _dpq(assign_logits, codebooks, tile_v=1000):
    logits2d = assign_logits.reshape(V, M * K)
    cb2d = codebooks.reshape(M * K, CHUNK)
    grid = (V // tile_v,)
    return pl.pallas_call(
        _dpq_body,
        grid=grid,
        in_specs=[
            pl.BlockSpec((tile_v, M * K), lambda i: (i, 0)),
            pl.BlockSpec((M * K, CHUNK), lambda i: (0, 0)),
        ],
        out_specs=pl.BlockSpec((tile_v, D), lambda i: (i, 0)),
        out_shape=jax.ShapeDtypeStruct((V, D), jnp.float32),
        compiler_params=pltpu.CompilerParams(
            dimension_semantics=("parallel",),
        ),
    )(logits2d, cb2d)


def kernel(assign_logits, codebooks):
    return _dpq(assign_logits, codebooks)
